# Initial kernel scaffold; baseline (speedup 1.0000x reference)
import jax
import jax.numpy as jnp
from jax.experimental import pallas as pl


def kernel(x, edge_index, edge_weight, W1, b1, W2, b2):
    raise NotImplementedError("write your pallas kernel here")


# speedup vs baseline: 29.5967x; 29.5967x over previous
import functools

import jax
import jax.numpy as jnp
from jax import lax
from jax.experimental import pallas as pl
from jax.experimental.pallas import tpu as pltpu
from jax.experimental.pallas import tpu_sc as plsc

N = 50000
E = 800000
DIN, DHID, DOUT = 64, 16, 32

NC, NS, L = 2, 16, 16
NW = NC * NS
EPT = 25600
EPAD = EPT * NW
CH = 1024
NCHUNK = EPT // CH
ROWS_PT = N // NS
ZROWS = 625
_BN = 2000
_NB = N // _BN

_f32 = jnp.float32
_i32 = jnp.int32

_mesh = plsc.VectorSubcoreMesh(core_axis_name="c", subcore_axis_name="s")
_sc_params = pltpu.CompilerParams(needs_layout_passes=False,
                                  use_tc_tiling_on_sc=False)


def _deg_body(col2d, ew, out, colbuf, ewbuf, degbuf):
    c = lax.axis_index("c")
    s = lax.axis_index("s")
    w = s * NC + c

    def zero(i, _):
        degbuf[pl.ds(i * L, L)] = jnp.zeros((L,), _f32)
        return 0

    lax.fori_loop(0, N // L, zero, 0)

    def chunk(ci, _):
        base8 = w * (EPT // 128) + ci * (CH // 128)
        base = w * EPT + ci * CH
        pltpu.sync_copy(col2d.at[pl.ds(base8, CH // 128)], colbuf)
        pltpu.sync_copy(ew.at[pl.ds(base, CH)], ewbuf)
        for j in range(CH // 128):
            for k in range(128 // L):
                cv = colbuf[j, pl.ds(k * L, L)]
                wv = ewbuf[pl.ds(j * 128 + k * L, L)]
                plsc.addupdate_scatter(degbuf, [cv], wv)
        return 0

    lax.fori_loop(0, NCHUNK, chunk, 0)
    for b in range(N // _BN):
        pltpu.sync_copy(degbuf.at[pl.ds(b * _BN, _BN)], out.at[b, w])


_deg_call = pl.kernel(
    _deg_body,
    out_type=jax.ShapeDtypeStruct((_NB, NW, _BN), _f32),
    mesh=_mesh,
    compiler_params=_sc_params,
    scratch_types=[
        pltpu.VMEM((CH // 128, 128), _i32),
        pltpu.VMEM((CH,), _f32),
        pltpu.VMEM((N,), _f32),
    ],
)


def _msg_body(row2d, col2d, ew, g, out, rowbuf, colbuf, ewbuf, rowsbuf, zbuf,
              sem, s_sh):
    c = lax.axis_index("c")
    s = lax.axis_index("s")
    w = s * NC + c

    def zero(i, _):
        zbuf[i, :] = jnp.zeros((L,), _f32)
        return 0

    lax.fori_loop(0, ZROWS, zero, 0)
    for k in range(ROWS_PT // ZROWS):
        pltpu.sync_copy(zbuf, s_sh.at[pl.ds(s * ROWS_PT + k * ZROWS, ZROWS)])
    plsc.subcore_barrier()

    def chunk(ci, _):
        base8 = w * (EPT // 128) + ci * (CH // 128)
        base = w * EPT + ci * CH
        pltpu.sync_copy(row2d.at[pl.ds(base8, CH // 128)], rowbuf)
        pltpu.sync_copy(col2d.at[pl.ds(base8, CH // 128)], colbuf)
        pltpu.sync_copy(ew.at[pl.ds(base, CH)], ewbuf)
        cps = [
            pltpu.async_copy(g.at[rowbuf.at[j]],
                             rowsbuf.at[pl.ds(j * 128, 128)], sem)
            for j in range(CH // 128)
        ]
        for cp in cps:
            cp.wait()

        def scale16(i, _):
            for j in range(L):
                e = i * L + j
                b = plsc.load_gather(ewbuf, [jnp.full((L,), e, _i32)])
                rowsbuf[e, :] = rowsbuf[e, :] * b
            return 0

        lax.fori_loop(0, CH // L, scale16, 0)
        for j in range(CH // 128):
            pltpu.sync_copy(rowsbuf.at[pl.ds(j * 128, 128)],
                            s_sh.at[colbuf.at[j]], add=True)
        return 0

    lax.fori_loop(0, NCHUNK, chunk, 0)
    plsc.subcore_barrier()
    pltpu.sync_copy(s_sh.at[pl.ds(s * ROWS_PT, ROWS_PT)],
                    out.at[c, pl.ds(s * ROWS_PT, ROWS_PT)])


_msg_call = pl.kernel(
    _msg_body,
    out_type=jax.ShapeDtypeStruct((NC, N, DHID), _f32),
    mesh=_mesh,
    compiler_params=_sc_params,
    scratch_types=[
        pltpu.VMEM((CH // 128, 128), _i32),
        pltpu.VMEM((CH // 128, 128), _i32),
        pltpu.VMEM((CH,), _f32),
        pltpu.VMEM((CH, DHID), _f32),
        pltpu.VMEM((ZROWS, DHID), _f32),
        pltpu.SemaphoreType.DMA,
        pltpu.VMEM_SHARED((N, DHID), _f32),
    ],
)


def _pre_body(degs_ref, x_ref, w1_ref, g_ref, hs_ref, dis_ref):
    deg = jnp.sum(degs_ref[0], axis=0) + 1.0
    dis = lax.rsqrt(deg)
    h = jnp.dot(x_ref[...], w1_ref[...], preferred_element_type=_f32)
    g_ref[...] = h * dis[:, None]
    hs_ref[...] = h * (dis * dis)[:, None]
    dis_ref[pl.program_id(0), :] = dis


_pre_call = pl.pallas_call(
    _pre_body,
    grid=(_NB,),
    in_specs=[
        pl.BlockSpec((1, NW, _BN), lambda i: (i, 0, 0)),
        pl.BlockSpec((_BN, DIN), lambda i: (i, 0)),
        pl.BlockSpec((DIN, DHID), lambda i: (0, 0)),
    ],
    out_specs=[
        pl.BlockSpec((_BN, DHID), lambda i: (i, 0)),
        pl.BlockSpec((_BN, DHID), lambda i: (i, 0)),
        pl.BlockSpec((_NB, _BN), lambda i: (0, 0)),
    ],
    out_shape=[
        jax.ShapeDtypeStruct((N, DHID), _f32),
        jax.ShapeDtypeStruct((N, DHID), _f32),
        jax.ShapeDtypeStruct((_NB, _BN), _f32),
    ],
)


def _fin_body(s_ref, hs_ref, dis_ref, b1_ref, w2_ref, b2_ref, out_ref):
    ssum = s_ref[0] + s_ref[1]
    dis = dis_ref[pl.program_id(0), :]
    agg = ssum * dis[:, None] + hs_ref[...] + b1_ref[0][None, :]
    emb = jnp.maximum(agg, 0.0)
    out_ref[...] = (jnp.dot(emb, w2_ref[...], preferred_element_type=_f32)
                    + b2_ref[0][None, :])


_fin_call = pl.pallas_call(
    _fin_body,
    grid=(_NB,),
    in_specs=[
        pl.BlockSpec((NC, _BN, DHID), lambda i: (0, i, 0)),
        pl.BlockSpec((_BN, DHID), lambda i: (i, 0)),
        pl.BlockSpec((_NB, _BN), lambda i: (0, 0)),
        pl.BlockSpec((1, DHID), lambda i: (0, 0)),
        pl.BlockSpec((DHID, DOUT), lambda i: (0, 0)),
        pl.BlockSpec((1, DOUT), lambda i: (0, 0)),
    ],
    out_specs=pl.BlockSpec((_BN, DOUT), lambda i: (i, 0)),
    out_shape=jax.ShapeDtypeStruct((N, DOUT), _f32),
)


def kernel(x, edge_index, edge_weight, W1, b1, W2, b2):
    pad = EPAD - E
    row2d = jnp.pad(edge_index[0], (0, pad)).reshape(EPAD // 128, 128)
    col2d = jnp.pad(edge_index[1], (0, pad)).reshape(EPAD // 128, 128)
    ewp = jnp.pad(edge_weight, (0, pad))

    degs = _deg_call(col2d, ewp)
    g, hs, dis = _pre_call(degs, x, W1)
    s2 = _msg_call(row2d, col2d, ewp, g)
    return _fin_call(s2, hs, dis, b1.reshape(1, DHID), W2,
                     b2.reshape(1, DOUT))


# speedup vs baseline: 41.1401x; 1.3900x over previous
import jax
import jax.numpy as jnp
from jax import lax
from jax.experimental import pallas as pl
from jax.experimental.pallas import tpu as pltpu
from jax.experimental.pallas import tpu_sc as plsc

N = 50000
E = 800000
DIN, DHID, DOUT = 64, 16, 32

NC, NS, L = 2, 16, 16
NW = NC * NS
CH = 1024
NCHUNK = 25
EPT = CH * NCHUNK
EPAD = EPT * NW
CH8 = CH // 128
ROWS_PT = N // NS
ZROWS = 625
_BN = 2000
_NB = N // _BN

_f32 = jnp.float32
_i32 = jnp.int32

_mesh = plsc.VectorSubcoreMesh(core_axis_name="c", subcore_axis_name="s")
_sc_params = pltpu.CompilerParams(needs_layout_passes=False,
                                  use_tc_tiling_on_sc=False)

_GDN = jax.lax.GatherDimensionNumbers(
    offset_dims=(), collapsed_slice_dims=(0,), start_index_map=(0,))


def _bcast(vec, j):
    return jax.lax.gather(
        vec, jnp.full((L, 1), j, _i32), _GDN, (1,),
        mode=jax.lax.GatherScatterMode.PROMISE_IN_BOUNDS)


def _deg_body(col2d, ew, out, cb0, cb1, eb0, eb1, degbuf, es0, es1, osem):
    c = lax.axis_index("c")
    s = lax.axis_index("s")
    w = s * NC + c
    CB = (cb0, cb1)
    EB = (eb0, eb1)
    ES = (es0, es1)

    def fetch(ci, k):
        base8 = w * (EPT // 128) + ci * CH8
        base = w * EPT + ci * CH
        pltpu.async_copy(col2d.at[pl.ds(base8, CH8)], CB[k], ES[k])
        pltpu.async_copy(ew.at[pl.ds(base, CH)], EB[k], ES[k])

    def wait_fetch(k):
        pltpu.make_async_copy(col2d.at[pl.ds(0, CH8)], CB[k], ES[k]).wait()
        pltpu.make_async_copy(ew.at[pl.ds(0, CH)], EB[k], ES[k]).wait()

    fetch(0, 0)

    def zero(i, _):
        degbuf[pl.ds(i * L, L)] = jnp.zeros((L,), _f32)
        return 0

    lax.fori_loop(0, N // L, zero, 0)

    for ci in range(NCHUNK):
        k = ci % 2
        wait_fetch(k)
        if ci + 1 < NCHUNK:
            fetch(ci + 1, 1 - k)
        for j in range(CH8):
            for m in range(128 // L):
                cv = CB[k][j, pl.ds(m * L, L)]
                wv = EB[k][pl.ds(j * 128 + m * L, L)]
                plsc.addupdate_scatter(degbuf, [cv], wv)

    cps = [pltpu.async_copy(degbuf.at[pl.ds(b * _BN, _BN)], out.at[b, w], osem)
           for b in range(N // _BN)]
    for cp in cps:
        cp.wait()


_deg_call = pl.kernel(
    _deg_body,
    out_type=jax.ShapeDtypeStruct((_NB, NW, _BN), _f32),
    mesh=_mesh,
    compiler_params=_sc_params,
    scratch_types=[
        pltpu.VMEM((CH8, 128), _i32),
        pltpu.VMEM((CH8, 128), _i32),
        pltpu.VMEM((CH,), _f32),
        pltpu.VMEM((CH,), _f32),
        pltpu.VMEM((N,), _f32),
        pltpu.SemaphoreType.DMA,
        pltpu.SemaphoreType.DMA,
        pltpu.SemaphoreType.DMA,
    ],
)


def _msg_body(row2d, col2d, ew, g, out,
              rb0, rb1, rb2, cb0, cb1, cb2, eb0, eb1, eb2,
              rs0, rs1, rs2, zbuf,
              es0, es1, es2, gs0, gs1, gs2, ss0, ss1, ss2, s_sh):
    c = lax.axis_index("c")
    s = lax.axis_index("s")
    w = s * NC + c
    RB = (rb0, rb1, rb2)
    CB = (cb0, cb1, cb2)
    EB = (eb0, eb1, eb2)
    RS = (rs0, rs1, rs2)
    ES = (es0, es1, es2)
    GS = (gs0, gs1, gs2)
    SS = (ss0, ss1, ss2)

    def fetch(ci, k):
        base8 = w * (EPT // 128) + ci * CH8
        base = w * EPT + ci * CH
        pltpu.async_copy(row2d.at[pl.ds(base8, CH8)], RB[k], ES[k])
        pltpu.async_copy(col2d.at[pl.ds(base8, CH8)], CB[k], ES[k])
        pltpu.async_copy(ew.at[pl.ds(base, CH)], EB[k], ES[k])

    def wait_fetch(k):
        pltpu.make_async_copy(row2d.at[pl.ds(0, CH8)], RB[k], ES[k]).wait()
        pltpu.make_async_copy(col2d.at[pl.ds(0, CH8)], CB[k], ES[k]).wait()
        pltpu.make_async_copy(ew.at[pl.ds(0, CH)], EB[k], ES[k]).wait()

    def gather(k):
        for j in range(CH8):
            pltpu.async_copy(g.at[RB[k].at[j]],
                             RS[k].at[pl.ds(j * 128, 128)], GS[k])

    def wait_gather(k):
        for j in range(CH8):
            pltpu.make_async_copy(g.at[RB[k].at[j]],
                                  RS[k].at[pl.ds(j * 128, 128)], GS[k]).wait()

    def scatter(k):
        for j in range(CH8):
            pltpu.async_copy(RS[k].at[pl.ds(j * 128, 128)],
                             s_sh.at[CB[k].at[j]], SS[k], add=True)

    def wait_scatter(k):
        for j in range(CH8):
            pltpu.make_async_copy(RS[k].at[pl.ds(j * 128, 128)],
                                  s_sh.at[CB[k].at[j]], SS[k]).wait()

    def scale(k):
        def s16(i, _):
            base = i * L
            ew16 = EB[k][pl.ds(base, L)]
            for j in range(L):
                e = base + j
                RS[k][e, :] = RS[k][e, :] * _bcast(ew16, j)
            return 0

        lax.fori_loop(0, CH // L, s16, 0)

    fetch(0, 0)
    fetch(1, 1)

    def zero(i, _):
        zbuf[i, :] = jnp.zeros((L,), _f32)
        return 0

    lax.fori_loop(0, ZROWS, zero, 0)
    zcps = [pltpu.async_copy(
        zbuf, s_sh.at[pl.ds(s * ROWS_PT + m * ZROWS, ZROWS)], ss0)
        for m in range(ROWS_PT // ZROWS)]
    for cp in zcps:
        cp.wait()
    plsc.subcore_barrier()

    for ci in range(NCHUNK):
        k = ci % 3
        wait_fetch(k)
        gather(k)
        wait_gather(k)
        scale(k)
        if ci >= 1:
            wait_scatter((k + 2) % 3)
        if ci + 2 < NCHUNK:
            fetch(ci + 2, (k + 2) % 3)
        scatter(k)
    wait_scatter((NCHUNK - 1) % 3)

    plsc.subcore_barrier()
    pltpu.sync_copy(s_sh.at[pl.ds(s * ROWS_PT, ROWS_PT)],
                    out.at[c, pl.ds(s * ROWS_PT, ROWS_PT)])


_msg_call = pl.kernel(
    _msg_body,
    out_type=jax.ShapeDtypeStruct((NC, N, DHID), _f32),
    mesh=_mesh,
    compiler_params=_sc_params,
    scratch_types=(
        [pltpu.VMEM((CH8, 128), _i32)] * 3
        + [pltpu.VMEM((CH8, 128), _i32)] * 3
        + [pltpu.VMEM((CH,), _f32)] * 3
        + [pltpu.VMEM((CH, DHID), _f32)] * 3
        + [pltpu.VMEM((ZROWS, DHID), _f32)]
        + [pltpu.SemaphoreType.DMA] * 9
        + [pltpu.VMEM_SHARED((N, DHID), _f32)]
    ),
)


def _h_body(x_ref, w1_ref, h_ref):
    h_ref[...] = jnp.dot(x_ref[...], w1_ref[...],
                         preferred_element_type=_f32)


_h_call = pl.pallas_call(
    _h_body,
    grid=(_NB,),
    in_specs=[
        pl.BlockSpec((_BN, DIN), lambda i: (i, 0)),
        pl.BlockSpec((DIN, DHID), lambda i: (0, 0)),
    ],
    out_specs=pl.BlockSpec((_BN, DHID), lambda i: (i, 0)),
    out_shape=jax.ShapeDtypeStruct((N, DHID), _f32),
)


def _pre_body(degs_ref, h_ref, g_ref, hs_ref, dis_ref):
    deg = jnp.sum(degs_ref[0], axis=0) + 1.0
    dis = lax.rsqrt(deg)
    h = h_ref[...]
    g_ref[...] = h * dis[:, None]
    hs_ref[...] = h * (dis * dis)[:, None]
    dis_ref[pl.program_id(0), :] = dis


_pre_call = pl.pallas_call(
    _pre_body,
    grid=(_NB,),
    in_specs=[
        pl.BlockSpec((1, NW, _BN), lambda i: (i, 0, 0)),
        pl.BlockSpec((_BN, DHID), lambda i: (i, 0)),
    ],
    out_specs=[
        pl.BlockSpec((_BN, DHID), lambda i: (i, 0)),
        pl.BlockSpec((_BN, DHID), lambda i: (i, 0)),
        pl.BlockSpec((_NB, _BN), lambda i: (0, 0)),
    ],
    out_shape=[
        jax.ShapeDtypeStruct((N, DHID), _f32),
        jax.ShapeDtypeStruct((N, DHID), _f32),
        jax.ShapeDtypeStruct((_NB, _BN), _f32),
    ],
)


def _fin_body(s_ref, hs_ref, dis_ref, b1_ref, w2_ref, b2_ref, out_ref):
    ssum = s_ref[0] + s_ref[1]
    dis = dis_ref[pl.program_id(0), :]
    agg = ssum * dis[:, None] + hs_ref[...] + b1_ref[0][None, :]
    emb = jnp.maximum(agg, 0.0)
    out_ref[...] = (jnp.dot(emb, w2_ref[...], preferred_element_type=_f32)
                    + b2_ref[0][None, :])


_fin_call = pl.pallas_call(
    _fin_body,
    grid=(_NB,),
    in_specs=[
        pl.BlockSpec((NC, _BN, DHID), lambda i: (0, i, 0)),
        pl.BlockSpec((_BN, DHID), lambda i: (i, 0)),
        pl.BlockSpec((_NB, _BN), lambda i: (0, 0)),
        pl.BlockSpec((1, DHID), lambda i: (0, 0)),
        pl.BlockSpec((DHID, DOUT), lambda i: (0, 0)),
        pl.BlockSpec((1, DOUT), lambda i: (0, 0)),
    ],
    out_specs=pl.BlockSpec((_BN, DOUT), lambda i: (i, 0)),
    out_shape=jax.ShapeDtypeStruct((N, DOUT), _f32),
)


def kernel(x, edge_index, edge_weight, W1, b1, W2, b2):
    pad = EPAD - E
    row2d = jnp.pad(edge_index[0], (0, pad)).reshape(EPAD // 128, 128)
    col2d = jnp.pad(edge_index[1], (0, pad)).reshape(EPAD // 128, 128)
    ewp = jnp.pad(edge_weight, (0, pad))

    degs = _deg_call(col2d, ewp)
    h = _h_call(x, W1)
    g, hs, dis = _pre_call(degs, h)
    s2 = _msg_call(row2d, col2d, ewp, g)
    return _fin_call(s2, hs, dis, b1.reshape(1, DHID), W2,
                     b2.reshape(1, DOUT))


# speedup vs baseline: 57.7962x; 1.4049x over previous
import jax
import jax.numpy as jnp
from jax import lax
from jax.experimental import pallas as pl
from jax.experimental.pallas import tpu as pltpu
from jax.experimental.pallas import tpu_sc as plsc

N = 50000
E = 800000
DIN, DHID, DOUT = 64, 16, 32

NC, NS, L = 2, 16, 16
NW = NC * NS
CH = 1024
NCHUNK = 25
EPT = CH * NCHUNK
EPAD = EPT * NW
CH8 = CH // 128
ROWS_PT = N // NS
ZROWS = 625
_BN = 2000
_NB = N // _BN

_f32 = jnp.float32
_i32 = jnp.int32

_mesh = plsc.VectorSubcoreMesh(core_axis_name="c", subcore_axis_name="s")
_sc_params = pltpu.CompilerParams(needs_layout_passes=False,
                                  use_tc_tiling_on_sc=False)

_GDN = jax.lax.GatherDimensionNumbers(
    offset_dims=(), collapsed_slice_dims=(0,), start_index_map=(0,))


def _bcast(vec, j):
    return jax.lax.gather(
        vec, jnp.full((L, 1), j, _i32), _GDN, (1,),
        mode=jax.lax.GatherScatterMode.PROMISE_IN_BOUNDS)


def _deg_body(col2d, ew, out, cb0, cb1, eb0, eb1, degbuf, es0, es1, osem):
    c = lax.axis_index("c")
    s = lax.axis_index("s")
    w = s * NC + c
    CB = (cb0, cb1)
    EB = (eb0, eb1)
    ES = (es0, es1)

    def fetch(ci, k):
        base8 = w * (EPT // 128) + ci * CH8
        base = w * EPT + ci * CH
        pltpu.async_copy(col2d.at[pl.ds(base8, CH8)], CB[k], ES[k])
        pltpu.async_copy(ew.at[pl.ds(base, CH)], EB[k], ES[k])

    def wait_fetch(k):
        pltpu.make_async_copy(col2d.at[pl.ds(0, CH8)], CB[k], ES[k]).wait()
        pltpu.make_async_copy(ew.at[pl.ds(0, CH)], EB[k], ES[k]).wait()

    fetch(0, 0)

    def zero(i, _):
        for j in range(8):
            degbuf[pl.ds((i * 8 + j) * L, L)] = jnp.zeros((L,), _f32)
        return 0

    lax.fori_loop(0, N // L // 8, zero, 0)
    for i in range(N // L // 8 * 8, N // L):
        degbuf[pl.ds(i * L, L)] = jnp.zeros((L,), _f32)

    for ci in range(NCHUNK):
        k = ci % 2
        wait_fetch(k)
        if ci + 1 < NCHUNK:
            fetch(ci + 1, 1 - k)
        for j in range(CH8):
            for m in range(128 // L):
                cv = CB[k][j, pl.ds(m * L, L)]
                wv = EB[k][pl.ds(j * 128 + m * L, L)]
                plsc.addupdate_scatter(degbuf, [cv], wv)

    cps = [pltpu.async_copy(degbuf.at[pl.ds(b * _BN, _BN)], out.at[b, w], osem)
           for b in range(N // _BN)]
    for cp in cps:
        cp.wait()


_deg_call = pl.kernel(
    _deg_body,
    out_type=jax.ShapeDtypeStruct((_NB, NW, _BN), _f32),
    mesh=_mesh,
    compiler_params=_sc_params,
    scratch_types=[
        pltpu.VMEM((CH8, 128), _i32),
        pltpu.VMEM((CH8, 128), _i32),
        pltpu.VMEM((CH,), _f32),
        pltpu.VMEM((CH,), _f32),
        pltpu.VMEM((N,), _f32),
        pltpu.SemaphoreType.DMA,
        pltpu.SemaphoreType.DMA,
        pltpu.SemaphoreType.DMA,
    ],
)


def _msg_body(row2d, col2d, ew, g, out,
              rb0, rb1, rb2, cb0, cb1, cb2, eb0, eb1, eb2,
              rs0, rs1, rs2, zbuf,
              es0, es1, es2, gs0, gs1, gs2, ss0, ss1, ss2, s_sh):
    c = lax.axis_index("c")
    s = lax.axis_index("s")
    w = s * NC + c
    RB = (rb0, rb1, rb2)
    CB = (cb0, cb1, cb2)
    EB = (eb0, eb1, eb2)
    RS = (rs0, rs1, rs2)
    ES = (es0, es1, es2)
    GS = (gs0, gs1, gs2)
    SS = (ss0, ss1, ss2)

    def fetch(ci, k):
        base8 = w * (EPT // 128) + ci * CH8
        base = w * EPT + ci * CH
        pltpu.async_copy(row2d.at[pl.ds(base8, CH8)], RB[k], ES[k])
        pltpu.async_copy(col2d.at[pl.ds(base8, CH8)], CB[k], ES[k])
        pltpu.async_copy(ew.at[pl.ds(base, CH)], EB[k], ES[k])

    def wait_fetch(k):
        pltpu.make_async_copy(row2d.at[pl.ds(0, CH8)], RB[k], ES[k]).wait()
        pltpu.make_async_copy(col2d.at[pl.ds(0, CH8)], CB[k], ES[k]).wait()
        pltpu.make_async_copy(ew.at[pl.ds(0, CH)], EB[k], ES[k]).wait()

    def gather(k):
        for j in range(CH8):
            pltpu.async_copy(g.at[RB[k].at[j]],
                             RS[k].at[pl.ds(j * 128, 128)], GS[k])

    def wait_gather(k):
        for j in range(CH8):
            pltpu.make_async_copy(g.at[RB[k].at[j]],
                                  RS[k].at[pl.ds(j * 128, 128)], GS[k]).wait()

    def scatter(k):
        for j in range(CH8):
            pltpu.async_copy(RS[k].at[pl.ds(j * 128, 128)],
                             s_sh.at[CB[k].at[j]], SS[k], add=True)

    def wait_scatter(k):
        for j in range(CH8):
            pltpu.make_async_copy(RS[k].at[pl.ds(j * 128, 128)],
                                  s_sh.at[CB[k].at[j]], SS[k]).wait()

    def scale(k):
        def s16(i, _):
            base = i * L
            ew16 = EB[k][pl.ds(base, L)]
            for j in range(L):
                e = base + j
                RS[k][e, :] = RS[k][e, :] * _bcast(ew16, j)
            return 0

        lax.fori_loop(0, CH // L, s16, 0)

    fetch(0, 0)
    fetch(1, 1)

    def zero(i, _):
        for j in range(8):
            zbuf[i * 8 + j, :] = jnp.zeros((L,), _f32)
        return 0

    lax.fori_loop(0, ZROWS // 8, zero, 0)
    for i in range(ZROWS // 8 * 8, ZROWS):
        zbuf[i, :] = jnp.zeros((L,), _f32)
    zcps = [pltpu.async_copy(
        zbuf, s_sh.at[pl.ds(s * ROWS_PT + m * ZROWS, ZROWS)], ss0)
        for m in range(ROWS_PT // ZROWS)]
    for cp in zcps:
        cp.wait()
    plsc.subcore_barrier()

    for ci in range(NCHUNK):
        k = ci % 3
        wait_fetch(k)
        gather(k)
        wait_gather(k)
        scale(k)
        if ci >= 1:
            wait_scatter((k + 2) % 3)
        if ci + 2 < NCHUNK:
            fetch(ci + 2, (k + 2) % 3)
        scatter(k)
    wait_scatter((NCHUNK - 1) % 3)

    plsc.subcore_barrier()
    pltpu.sync_copy(s_sh.at[pl.ds(s * ROWS_PT, ROWS_PT)],
                    out.at[c, pl.ds(s * ROWS_PT, ROWS_PT)])


_msg_call = pl.kernel(
    _msg_body,
    out_type=jax.ShapeDtypeStruct((NC, N, DHID), _f32),
    mesh=_mesh,
    compiler_params=_sc_params,
    scratch_types=(
        [pltpu.VMEM((CH8, 128), _i32)] * 3
        + [pltpu.VMEM((CH8, 128), _i32)] * 3
        + [pltpu.VMEM((CH,), _f32)] * 3
        + [pltpu.VMEM((CH, DHID), _f32)] * 3
        + [pltpu.VMEM((ZROWS, DHID), _f32)]
        + [pltpu.SemaphoreType.DMA] * 9
        + [pltpu.VMEM_SHARED((N, DHID), _f32)]
    ),
)


def _h_body(x_ref, w1_ref, h_ref):
    h_ref[...] = jnp.dot(x_ref[...], w1_ref[...],
                         preferred_element_type=_f32)


_h_call = pl.pallas_call(
    _h_body,
    grid=(_NB,),
    in_specs=[
        pl.BlockSpec((_BN, DIN), lambda i: (i, 0)),
        pl.BlockSpec((DIN, DHID), lambda i: (0, 0)),
    ],
    out_specs=pl.BlockSpec((_BN, DHID), lambda i: (i, 0)),
    out_shape=jax.ShapeDtypeStruct((N, DHID), _f32),
)


def _pre_body(degs_ref, h_ref, g_ref, hs_ref, dis_ref):
    deg = jnp.sum(degs_ref[0], axis=0) + 1.0
    dis = lax.rsqrt(deg)
    h = h_ref[...]
    g_ref[...] = h * dis[:, None]
    hs_ref[...] = h * (dis * dis)[:, None]
    dis_ref[pl.program_id(0), :] = dis


_pre_call = pl.pallas_call(
    _pre_body,
    grid=(_NB,),
    in_specs=[
        pl.BlockSpec((1, NW, _BN), lambda i: (i, 0, 0)),
        pl.BlockSpec((_BN, DHID), lambda i: (i, 0)),
    ],
    out_specs=[
        pl.BlockSpec((_BN, DHID), lambda i: (i, 0)),
        pl.BlockSpec((_BN, DHID), lambda i: (i, 0)),
        pl.BlockSpec((_NB, _BN), lambda i: (0, 0)),
    ],
    out_shape=[
        jax.ShapeDtypeStruct((N, DHID), _f32),
        jax.ShapeDtypeStruct((N, DHID), _f32),
        jax.ShapeDtypeStruct((_NB, _BN), _f32),
    ],
)


def _fin_body(s_ref, hs_ref, dis_ref, b1_ref, w2_ref, b2_ref, out_ref):
    ssum = s_ref[0] + s_ref[1]
    dis = dis_ref[pl.program_id(0), :]
    agg = ssum * dis[:, None] + hs_ref[...] + b1_ref[0][None, :]
    emb = jnp.maximum(agg, 0.0)
    out_ref[...] = (jnp.dot(emb, w2_ref[...], preferred_element_type=_f32)
                    + b2_ref[0][None, :])


_fin_call = pl.pallas_call(
    _fin_body,
    grid=(_NB,),
    in_specs=[
        pl.BlockSpec((NC, _BN, DHID), lambda i: (0, i, 0)),
        pl.BlockSpec((_BN, DHID), lambda i: (i, 0)),
        pl.BlockSpec((_NB, _BN), lambda i: (0, 0)),
        pl.BlockSpec((1, DHID), lambda i: (0, 0)),
        pl.BlockSpec((DHID, DOUT), lambda i: (0, 0)),
        pl.BlockSpec((1, DOUT), lambda i: (0, 0)),
    ],
    out_specs=pl.BlockSpec((_BN, DOUT), lambda i: (i, 0)),
    out_shape=jax.ShapeDtypeStruct((N, DOUT), _f32),
)


def kernel(x, edge_index, edge_weight, W1, b1, W2, b2):
    pad = EPAD - E
    pad_idx = jnp.arange(pad, dtype=_i32) % N
    row2d = jnp.concatenate([edge_index[0], pad_idx]).reshape(EPAD // 128, 128)
    col2d = jnp.concatenate([edge_index[1], pad_idx]).reshape(EPAD // 128, 128)
    ewp = jnp.pad(edge_weight, (0, pad))

    degs = _deg_call(col2d, ewp)
    h = _h_call(x, W1)
    g, hs, dis = _pre_call(degs, h)
    s2 = _msg_call(row2d, col2d, ewp, g)
    return _fin_call(s2, hs, dis, b1.reshape(1, DHID), W2,
                     b2.reshape(1, DOUT))
